# baseline (speedup 1.0000x reference)
import functools

import jax
import jax.numpy as jnp
from jax import lax
from jax.experimental import pallas as pl
from jax.experimental.pallas import tpu as pltpu
from jax.experimental.pallas import tpu_sc as plsc

DIM = 32
BAG = 20
BC = 64
ROWS = BAG * BC
IDX_MINOR = 128
IDX_ROWS = ROWS // IDX_MINOR


NCH = 800


@functools.partial(jax.jit, static_argnums=(1, 2, 3))
def _sc_transpose(wt, N, T, num_workers):
    units = T * (N // NCH)
    upw = -(-units // num_workers)
    mesh = plsc.VectorSubcoreMesh(core_axis_name="c", subcore_axis_name="s")

    @functools.partial(
        pl.kernel,
        mesh=mesh,
        compiler_params=pltpu.CompilerParams(
            use_tc_tiling_on_sc=False, needs_layout_passes=False
        ),
        out_type=jax.ShapeDtypeStruct((T * N, DIM), jnp.float32),
        scratch_types=[
            pltpu.VMEM((2, DIM, NCH), jnp.float32),
            pltpu.VMEM((2, NCH, DIM), jnp.float32),
            pltpu.SemaphoreType.DMA,
            pltpu.SemaphoreType.DMA,
        ],
    )
    def k(wt_hbm, out_hbm, slab_v, tr_v, sem0, sem1):
        sems = (sem0, sem1)
        wid = lax.axis_index("s") * 2 + lax.axis_index("c")
        u0 = wid * upw
        nchunks = N // NCH
        lane = jax.lax.iota(jnp.int32, 16)
        cols = [jnp.full((16,), d, jnp.int32) for d in range(DIM)]

        def start(u, b):
            t = u // nchunks
            n0 = (u % nchunks) * NCH
            pltpu.async_copy(
                wt_hbm.at[t, :, pl.ds(n0, NCH)], slab_v.at[b], sems[b]
            )

        def wait(b):
            pltpu.make_async_copy(
                wt_hbm.at[0, :, pl.ds(0, NCH)], slab_v.at[b], sems[b]
            ).wait()

        def process(u, b):
            def grp(g, carry):
                rows = lane + g * 16
                for d in range(DIM):
                    v = slab_v[b, d, pl.ds(g * 16, 16)]
                    plsc.store_scatter(tr_v.at[b], [rows, cols[d]], v)
                return carry

            lax.fori_loop(0, NCH // 16, grp, 0)
            t = u // nchunks
            n0 = (u % nchunks) * NCH
            pltpu.sync_copy(tr_v.at[b], out_hbm.at[pl.ds(t * N + n0, NCH), :])

        @pl.when(u0 < units)
        def _():
            start(u0, 0)

        def pair_body(g, carry):
            for bpar in range(2):
                u = u0 + 2 * g + bpar
                nxt = 1 - bpar

                @pl.when(u + 1 < jnp.minimum(u0 + upw, units))
                def _():
                    start(u + 1, nxt)

                @pl.when(u < units)
                def _():
                    wait(bpar)
                    process(u, bpar)
            return carry

        lax.fori_loop(0, upw // 2, pair_body, 0)

    return k(wt)


@functools.partial(jax.jit, static_argnums=(2, 3, 4, 5))
def _sc_lookup(table, idx_t, N, T, B, num_workers):
    units_per_worker = (T * B // BC) // num_workers
    mesh = plsc.VectorSubcoreMesh(core_axis_name="c", subcore_axis_name="s")

    @functools.partial(
        pl.kernel,
        mesh=mesh,
        compiler_params=pltpu.CompilerParams(
            use_tc_tiling_on_sc=False, needs_layout_passes=False
        ),
        out_type=jax.ShapeDtypeStruct((T, DIM, B), jnp.float32),
        scratch_types=[
            pltpu.VMEM((2, BAG, BC), jnp.int32),
            pltpu.VMEM((2, IDX_ROWS, IDX_MINOR), jnp.int32),
            pltpu.VMEM((2, ROWS, DIM), jnp.float32),
            pltpu.VMEM((DIM, BC), jnp.float32),
            pltpu.SemaphoreType.DMA,
            pltpu.SemaphoreType.DMA,
        ],
    )
    def k(table_hbm, idx_hbm, out_hbm, idx_v, flat_v, rows_v, out_v, sem0, sem1):
        table2d = table_hbm
        sems = (sem0, sem1)
        wid = lax.axis_index("s") * 2 + lax.axis_index("c")
        b0s = (wid * 2 * BC, (wid * 2 + 1) * BC)

        def start(t, sub, b):
            pltpu.sync_copy(idx_hbm.at[t, :, pl.ds(b0s[sub], BC)], idx_v.at[b])
            for kk in range(ROWS // 16):
                v = idx_v[b, kk // 4, pl.ds((kk % 4) * 16, 16)]
                flat_v[b, kk // 8, pl.ds((kk % 8) * 16, 16)] = v + t * N
            for j in range(IDX_ROWS):
                pltpu.async_copy(
                    table2d.at[flat_v.at[b, j]],
                    rows_v.at[b, pl.ds(j * IDX_MINOR, IDX_MINOR)],
                    sems[b],
                )

        def wait(b):
            for j in range(IDX_ROWS):
                pltpu.make_async_copy(
                    table2d.at[flat_v.at[b, j]],
                    rows_v.at[b, pl.ds(j * IDX_MINOR, IDX_MINOR)],
                    sems[b],
                ).wait()

        lane = jax.lax.iota(jnp.int32, 16)
        row_lo = lane
        row_hi = lane + 16

        def reduce_store(t, sub, b):
            def bag_body(bag, carry):
                a0 = rows_v[b, bag, pl.ds(0, 16)]
                a1 = rows_v[b, bag, pl.ds(16, 16)]
                for l in range(1, BAG):
                    a0 = a0 + rows_v[b, bag + l * BC, pl.ds(0, 16)]
                    a1 = a1 + rows_v[b, bag + l * BC, pl.ds(16, 16)]
                col = jnp.full((16,), 0, jnp.int32) + bag
                plsc.store_scatter(out_v, [row_lo, col], a0)
                plsc.store_scatter(out_v, [row_hi, col], a1)
                return carry

            lax.fori_loop(0, BC, bag_body, 0)
            pltpu.sync_copy(out_v, out_hbm.at[t, :, pl.ds(b0s[sub], BC)])

        start(0, 0, 0)

        def pair_body(g, carry):
            for bpar in range(2):
                u = 2 * g + bpar
                nxt = 1 - bpar

                @pl.when(u + 1 < units_per_worker)
                def _():
                    start(g + bpar, nxt, nxt)

                wait(bpar)
                reduce_store(g, bpar, bpar)
            return carry

        lax.fori_loop(0, units_per_worker // 2, pair_body, 0)

    return k(table, idx_t)


def kernel(embedding_weights, sharded_sparse_features):
    N, T, D = embedding_weights.shape
    B, _, L = sharded_sparse_features.shape
    wt = embedding_weights.transpose(1, 2, 0)
    table = _sc_transpose(wt, N, T, 32)
    idx_t = sharded_sparse_features.astype(jnp.int32).transpose(1, 2, 0)
    out = _sc_lookup(table, idx_t, N, T, B, 32)
    return out.transpose(2, 0, 1)


# speedup vs baseline: 1.1857x; 1.1857x over previous
import functools

import jax
import jax.numpy as jnp
from jax import lax
from jax.experimental import pallas as pl
from jax.experimental.pallas import tpu as pltpu
from jax.experimental.pallas import tpu_sc as plsc

DIM = 32
BAG = 20
BC = 64
ROWS = BAG * BC
IDX_MINOR = 128
IDX_ROWS = ROWS // IDX_MINOR


NCH = 800


@functools.partial(jax.jit, static_argnums=(1, 2, 3))
def _sc_transpose(wt, N, T, num_workers):
    units = T * (N // NCH)
    upw = -(-units // num_workers)
    mesh = plsc.VectorSubcoreMesh(core_axis_name="c", subcore_axis_name="s")

    @functools.partial(
        pl.kernel,
        mesh=mesh,
        compiler_params=pltpu.CompilerParams(
            use_tc_tiling_on_sc=False, needs_layout_passes=False
        ),
        out_type=jax.ShapeDtypeStruct((T * N, DIM), jnp.float32),
        scratch_types=[
            pltpu.VMEM((2, DIM, NCH), jnp.float32),
            pltpu.VMEM((2, NCH, DIM), jnp.float32),
            pltpu.SemaphoreType.DMA,
            pltpu.SemaphoreType.DMA,
        ],
    )
    def k(wt_hbm, out_hbm, slab_v, tr_v, sem0, sem1):
        sems = (sem0, sem1)
        wid = lax.axis_index("s") * 2 + lax.axis_index("c")
        u0 = wid * upw
        nchunks = N // NCH
        lane = jax.lax.iota(jnp.int32, 16)
        cols = [jnp.full((16,), d, jnp.int32) for d in range(DIM)]

        def start(u, b):
            t = u // nchunks
            n0 = (u % nchunks) * NCH
            pltpu.async_copy(
                wt_hbm.at[t, :, pl.ds(n0, NCH)], slab_v.at[b], sems[b]
            )

        def wait(b):
            pltpu.make_async_copy(
                wt_hbm.at[0, :, pl.ds(0, NCH)], slab_v.at[b], sems[b]
            ).wait()

        def process(u, b):
            def grp(g, carry):
                rows = lane + g * 16
                vs = [slab_v[b, d, pl.ds(g * 16, 16)] for d in range(DIM)]
                for d in range(DIM):
                    plsc.store_scatter(tr_v.at[b], [rows, cols[d]], vs[d])
                return carry

            lax.fori_loop(0, NCH // 16, grp, 0)
            t = u // nchunks
            n0 = (u % nchunks) * NCH
            pltpu.sync_copy(tr_v.at[b], out_hbm.at[pl.ds(t * N + n0, NCH), :])

        @pl.when(u0 < units)
        def _():
            start(u0, 0)

        def pair_body(g, carry):
            for bpar in range(2):
                u = u0 + 2 * g + bpar
                nxt = 1 - bpar

                @pl.when(u + 1 < jnp.minimum(u0 + upw, units))
                def _():
                    start(u + 1, nxt)

                @pl.when(u < units)
                def _():
                    wait(bpar)
                    process(u, bpar)
            return carry

        lax.fori_loop(0, upw // 2, pair_body, 0)

    return k(wt)


@functools.partial(jax.jit, static_argnums=(2, 3, 4, 5))
def _sc_lookup(table, idx_t, N, T, B, num_workers):
    units_per_worker = (T * B // BC) // num_workers
    mesh = plsc.VectorSubcoreMesh(core_axis_name="c", subcore_axis_name="s")

    @functools.partial(
        pl.kernel,
        mesh=mesh,
        compiler_params=pltpu.CompilerParams(
            use_tc_tiling_on_sc=False, needs_layout_passes=False
        ),
        out_type=jax.ShapeDtypeStruct((T, DIM, B), jnp.float32),
        scratch_types=[
            pltpu.VMEM((2, BAG, BC), jnp.int32),
            pltpu.VMEM((2, IDX_ROWS, IDX_MINOR), jnp.int32),
            pltpu.VMEM((2, ROWS, DIM), jnp.float32),
            pltpu.VMEM((DIM, BC), jnp.float32),
            pltpu.SemaphoreType.DMA,
            pltpu.SemaphoreType.DMA,
        ],
    )
    def k(table_hbm, idx_hbm, out_hbm, idx_v, flat_v, rows_v, out_v, sem0, sem1):
        table2d = table_hbm
        sems = (sem0, sem1)
        wid = lax.axis_index("s") * 2 + lax.axis_index("c")
        b0s = (wid * 2 * BC, (wid * 2 + 1) * BC)

        def start(t, sub, b):
            pltpu.sync_copy(idx_hbm.at[t, :, pl.ds(b0s[sub], BC)], idx_v.at[b])
            for kk in range(ROWS // 16):
                v = idx_v[b, kk // 4, pl.ds((kk % 4) * 16, 16)]
                flat_v[b, kk // 8, pl.ds((kk % 8) * 16, 16)] = v + t * N
            for j in range(IDX_ROWS):
                pltpu.async_copy(
                    table2d.at[flat_v.at[b, j]],
                    rows_v.at[b, pl.ds(j * IDX_MINOR, IDX_MINOR)],
                    sems[b],
                )

        def wait(b):
            for j in range(IDX_ROWS):
                pltpu.make_async_copy(
                    table2d.at[flat_v.at[b, j]],
                    rows_v.at[b, pl.ds(j * IDX_MINOR, IDX_MINOR)],
                    sems[b],
                ).wait()

        lane = jax.lax.iota(jnp.int32, 16)
        row_lo = lane
        row_hi = lane + 16

        def reduce_store(t, sub, b):
            def bag_body(bag, carry):
                a0 = rows_v[b, bag, pl.ds(0, 16)]
                a1 = rows_v[b, bag, pl.ds(16, 16)]
                for l in range(1, BAG):
                    a0 = a0 + rows_v[b, bag + l * BC, pl.ds(0, 16)]
                    a1 = a1 + rows_v[b, bag + l * BC, pl.ds(16, 16)]
                col = jnp.full((16,), 0, jnp.int32) + bag
                plsc.store_scatter(out_v, [row_lo, col], a0)
                plsc.store_scatter(out_v, [row_hi, col], a1)
                return carry

            lax.fori_loop(0, BC, bag_body, 0)
            pltpu.sync_copy(out_v, out_hbm.at[t, :, pl.ds(b0s[sub], BC)])

        start(0, 0, 0)

        def pair_body(g, carry):
            for bpar in range(2):
                u = 2 * g + bpar
                nxt = 1 - bpar

                @pl.when(u + 1 < units_per_worker)
                def _():
                    start(g + bpar, nxt, nxt)

                wait(bpar)
                reduce_store(g, bpar, bpar)
            return carry

        lax.fori_loop(0, units_per_worker // 2, pair_body, 0)

    return k(table, idx_t)


def kernel(embedding_weights, sharded_sparse_features):
    N, T, D = embedding_weights.shape
    B, _, L = sharded_sparse_features.shape
    wt = embedding_weights.transpose(1, 2, 0)
    table = _sc_transpose(wt, N, T, 32)
    idx_t = sharded_sparse_features.astype(jnp.int32).transpose(1, 2, 0)
    out = _sc_lookup(table, idx_t, N, T, B, 32)
    return out.transpose(2, 0, 1)


# speedup vs baseline: 1.7596x; 1.4841x over previous
import functools

import jax
import jax.numpy as jnp
from jax import lax
from jax.experimental import pallas as pl
from jax.experimental.pallas import tpu as pltpu
from jax.experimental.pallas import tpu_sc as plsc

DIM = 32
BAG = 20
BC = 64
ROWS = BAG * BC
IDX_MINOR = 128
IDX_ROWS = ROWS // IDX_MINOR


NCH = 800


@functools.partial(jax.jit, static_argnums=(1, 2, 3))
def _sc_transpose(wt, N, T, num_workers):
    units = T * (N // NCH)
    upw = -(-units // num_workers)
    mesh = plsc.VectorSubcoreMesh(core_axis_name="c", subcore_axis_name="s")

    @functools.partial(
        pl.kernel,
        mesh=mesh,
        compiler_params=pltpu.CompilerParams(
            use_tc_tiling_on_sc=False, needs_layout_passes=False
        ),
        out_type=jax.ShapeDtypeStruct((T * N, DIM), jnp.float32),
        scratch_types=[
            pltpu.VMEM((2, DIM, NCH), jnp.float32),
            pltpu.VMEM((2, NCH, DIM + 1), jnp.float32),
            pltpu.SemaphoreType.DMA,
            pltpu.SemaphoreType.DMA,
        ],
    )
    def k(wt_hbm, out_hbm, slab_v, tr_v, sem0, sem1):
        sems = (sem0, sem1)
        wid = lax.axis_index("s") * 2 + lax.axis_index("c")
        u0 = wid * upw
        nchunks = N // NCH
        lane = jax.lax.iota(jnp.int32, 16)
        cols = [jnp.full((16,), d, jnp.int32) for d in range(DIM)]

        def start(u, b):
            t = u // nchunks
            n0 = (u % nchunks) * NCH
            pltpu.async_copy(
                wt_hbm.at[t, :, pl.ds(n0, NCH)], slab_v.at[b], sems[b]
            )

        def wait(b):
            pltpu.make_async_copy(
                wt_hbm.at[0, :, pl.ds(0, NCH)], slab_v.at[b], sems[b]
            ).wait()

        def process(u, b):
            def grp(g, carry):
                rows = lane + g * 16
                vs = [slab_v[b, d, pl.ds(g * 16, 16)] for d in range(DIM)]
                for d in range(DIM):
                    plsc.store_scatter(tr_v.at[b], [rows, cols[d]], vs[d])
                return carry

            lax.fori_loop(0, NCH // 16, grp, 0)
            t = u // nchunks
            n0 = (u % nchunks) * NCH
            pltpu.sync_copy(
                tr_v.at[b, :, pl.ds(0, DIM)],
                out_hbm.at[pl.ds(t * N + n0, NCH), :],
            )

        @pl.when(u0 < units)
        def _():
            start(u0, 0)

        def pair_body(g, carry):
            for bpar in range(2):
                u = u0 + 2 * g + bpar
                nxt = 1 - bpar

                @pl.when(u + 1 < jnp.minimum(u0 + upw, units))
                def _():
                    start(u + 1, nxt)

                @pl.when(u < units)
                def _():
                    wait(bpar)
                    process(u, bpar)
            return carry

        lax.fori_loop(0, upw // 2, pair_body, 0)

    return k(wt)


@functools.partial(jax.jit, static_argnums=(2, 3, 4, 5))
def _sc_lookup(table, idx_t, N, T, B, num_workers):
    units_per_worker = (T * B // BC) // num_workers
    mesh = plsc.VectorSubcoreMesh(core_axis_name="c", subcore_axis_name="s")

    @functools.partial(
        pl.kernel,
        mesh=mesh,
        compiler_params=pltpu.CompilerParams(
            use_tc_tiling_on_sc=False, needs_layout_passes=False
        ),
        out_type=jax.ShapeDtypeStruct((T, DIM, B), jnp.float32),
        scratch_types=[
            pltpu.VMEM((2, BAG, BC), jnp.int32),
            pltpu.VMEM((2, IDX_ROWS, IDX_MINOR), jnp.int32),
            pltpu.VMEM((2, ROWS, DIM), jnp.float32),
            pltpu.VMEM((DIM, BC + 1), jnp.float32),
            pltpu.SemaphoreType.DMA,
            pltpu.SemaphoreType.DMA,
        ],
    )
    def k(table_hbm, idx_hbm, out_hbm, idx_v, flat_v, rows_v, out_v, sem0, sem1):
        table2d = table_hbm
        sems = (sem0, sem1)
        wid = lax.axis_index("s") * 2 + lax.axis_index("c")
        b0s = (wid * 2 * BC, (wid * 2 + 1) * BC)

        def start(t, sub, b):
            pltpu.sync_copy(idx_hbm.at[t, :, pl.ds(b0s[sub], BC)], idx_v.at[b])
            for kk in range(ROWS // 16):
                v = idx_v[b, kk // 4, pl.ds((kk % 4) * 16, 16)]
                flat_v[b, kk // 8, pl.ds((kk % 8) * 16, 16)] = v + t * N
            for j in range(IDX_ROWS):
                pltpu.async_copy(
                    table2d.at[flat_v.at[b, j]],
                    rows_v.at[b, pl.ds(j * IDX_MINOR, IDX_MINOR)],
                    sems[b],
                )

        def wait(b):
            for j in range(IDX_ROWS):
                pltpu.make_async_copy(
                    table2d.at[flat_v.at[b, j]],
                    rows_v.at[b, pl.ds(j * IDX_MINOR, IDX_MINOR)],
                    sems[b],
                ).wait()

        lane = jax.lax.iota(jnp.int32, 16)
        row_lo = lane
        row_hi = lane + 16

        def reduce_store(t, sub, b):
            def bag_body(bag, carry):
                a0 = rows_v[b, bag, pl.ds(0, 16)]
                a1 = rows_v[b, bag, pl.ds(16, 16)]
                for l in range(1, BAG):
                    a0 = a0 + rows_v[b, bag + l * BC, pl.ds(0, 16)]
                    a1 = a1 + rows_v[b, bag + l * BC, pl.ds(16, 16)]
                col = jnp.full((16,), 0, jnp.int32) + bag
                plsc.store_scatter(out_v, [row_lo, col], a0)
                plsc.store_scatter(out_v, [row_hi, col], a1)
                return carry

            lax.fori_loop(0, BC, bag_body, 0)
            pltpu.sync_copy(
                out_v.at[:, pl.ds(0, BC)],
                out_hbm.at[t, :, pl.ds(b0s[sub], BC)],
            )

        start(0, 0, 0)

        def pair_body(g, carry):
            for bpar in range(2):
                u = 2 * g + bpar
                nxt = 1 - bpar

                @pl.when(u + 1 < units_per_worker)
                def _():
                    start(g + bpar, nxt, nxt)

                wait(bpar)
                reduce_store(g, bpar, bpar)
            return carry

        lax.fori_loop(0, units_per_worker // 2, pair_body, 0)

    return k(table, idx_t)


def kernel(embedding_weights, sharded_sparse_features):
    N, T, D = embedding_weights.shape
    B, _, L = sharded_sparse_features.shape
    wt = embedding_weights.transpose(1, 2, 0)
    table = _sc_transpose(wt, N, T, 32)
    idx_t = sharded_sparse_features.astype(jnp.int32).transpose(1, 2, 0)
    out = _sc_lookup(table, idx_t, N, T, B, 32)
    return out.transpose(2, 0, 1)


# speedup vs baseline: 1.7680x; 1.0047x over previous
import functools

import jax
import jax.numpy as jnp
from jax import lax
from jax.experimental import pallas as pl
from jax.experimental.pallas import tpu as pltpu
from jax.experimental.pallas import tpu_sc as plsc

DIM = 32
BAG = 20
BC = 64
ROWS = BAG * BC
IDX_MINOR = 128
IDX_ROWS = ROWS // IDX_MINOR


NCH = 800


@functools.partial(jax.jit, static_argnums=(1, 2, 3))
def _sc_transpose(wt, N, T, num_workers):
    units = T * (N // NCH)
    upw = -(-units // num_workers)
    mesh = plsc.VectorSubcoreMesh(core_axis_name="c", subcore_axis_name="s")

    @functools.partial(
        pl.kernel,
        mesh=mesh,
        compiler_params=pltpu.CompilerParams(
            use_tc_tiling_on_sc=False, needs_layout_passes=False
        ),
        out_type=jax.ShapeDtypeStruct((T * N, DIM), jnp.float32),
        scratch_types=[
            pltpu.VMEM((2, DIM, NCH), jnp.float32),
            pltpu.VMEM((2, NCH, DIM + 1), jnp.float32),
            pltpu.SemaphoreType.DMA,
            pltpu.SemaphoreType.DMA,
            pltpu.SemaphoreType.DMA,
            pltpu.SemaphoreType.DMA,
        ],
    )
    def k(wt_hbm, out_hbm, slab_v, tr_v, sem0, sem1, osem0, osem1):
        sems = (sem0, sem1)
        osems = (osem0, osem1)
        wid = lax.axis_index("s") * 2 + lax.axis_index("c")
        u0 = wid * upw
        nchunks = N // NCH
        lane = jax.lax.iota(jnp.int32, 16)
        cols = [jnp.full((16,), d, jnp.int32) for d in range(DIM)]

        def start(u, b):
            t = u // nchunks
            n0 = (u % nchunks) * NCH
            pltpu.async_copy(
                wt_hbm.at[t, :, pl.ds(n0, NCH)], slab_v.at[b], sems[b]
            )

        def wait(b):
            pltpu.make_async_copy(
                wt_hbm.at[0, :, pl.ds(0, NCH)], slab_v.at[b], sems[b]
            ).wait()

        def owait(b):
            pltpu.make_async_copy(
                tr_v.at[b, :, pl.ds(0, DIM)],
                out_hbm.at[pl.ds(0, NCH), :],
                osems[b],
            ).wait()

        def process(u, b):
            @pl.when(u >= u0 + 2)
            def _():
                owait(b)

            def grp(g, carry):
                rows = lane + g * 16
                vs = [slab_v[b, d, pl.ds(g * 16, 16)] for d in range(DIM)]
                for d in range(DIM):
                    plsc.store_scatter(tr_v.at[b], [rows, cols[d]], vs[d])
                return carry

            lax.fori_loop(0, NCH // 16, grp, 0)
            t = u // nchunks
            n0 = (u % nchunks) * NCH
            pltpu.async_copy(
                tr_v.at[b, :, pl.ds(0, DIM)],
                out_hbm.at[pl.ds(t * N + n0, NCH), :],
                osems[b],
            )

        @pl.when(u0 < units)
        def _():
            start(u0, 0)

        def pair_body(g, carry):
            for bpar in range(2):
                u = u0 + 2 * g + bpar
                nxt = 1 - bpar

                @pl.when(u + 1 < jnp.minimum(u0 + upw, units))
                def _():
                    start(u + 1, nxt)

                @pl.when(u < units)
                def _():
                    wait(bpar)
                    process(u, bpar)
            return carry

        lax.fori_loop(0, upw // 2, pair_body, 0)

        owait(0)
        owait(1)

    return k(wt)


@functools.partial(jax.jit, static_argnums=(2, 3, 4, 5))
def _sc_lookup(table, idx_t, N, T, B, num_workers):
    units_per_worker = (T * B // BC) // num_workers
    mesh = plsc.VectorSubcoreMesh(core_axis_name="c", subcore_axis_name="s")

    @functools.partial(
        pl.kernel,
        mesh=mesh,
        compiler_params=pltpu.CompilerParams(
            use_tc_tiling_on_sc=False, needs_layout_passes=False
        ),
        out_type=jax.ShapeDtypeStruct((T, DIM, B), jnp.float32),
        scratch_types=[
            pltpu.VMEM((2, BAG, BC), jnp.int32),
            pltpu.VMEM((2, IDX_ROWS, IDX_MINOR), jnp.int32),
            pltpu.VMEM((2, ROWS, DIM), jnp.float32),
            pltpu.VMEM((2, DIM, BC + 1), jnp.float32),
            pltpu.SemaphoreType.DMA,
            pltpu.SemaphoreType.DMA,
            pltpu.SemaphoreType.DMA,
            pltpu.SemaphoreType.DMA,
        ],
    )
    def k(table_hbm, idx_hbm, out_hbm, idx_v, flat_v, rows_v, out_v, sem0, sem1, osem0, osem1):
        table2d = table_hbm
        sems = (sem0, sem1)
        osems = (osem0, osem1)
        wid = lax.axis_index("s") * 2 + lax.axis_index("c")
        b0s = (wid * 2 * BC, (wid * 2 + 1) * BC)

        def start(t, sub, b):
            pltpu.sync_copy(idx_hbm.at[t, :, pl.ds(b0s[sub], BC)], idx_v.at[b])
            for kk in range(ROWS // 16):
                v = idx_v[b, kk // 4, pl.ds((kk % 4) * 16, 16)]
                flat_v[b, kk // 8, pl.ds((kk % 8) * 16, 16)] = v + t * N
            for j in range(IDX_ROWS):
                pltpu.async_copy(
                    table2d.at[flat_v.at[b, j]],
                    rows_v.at[b, pl.ds(j * IDX_MINOR, IDX_MINOR)],
                    sems[b],
                )

        def wait(b):
            for j in range(IDX_ROWS):
                pltpu.make_async_copy(
                    table2d.at[flat_v.at[b, j]],
                    rows_v.at[b, pl.ds(j * IDX_MINOR, IDX_MINOR)],
                    sems[b],
                ).wait()

        lane = jax.lax.iota(jnp.int32, 16)
        row_lo = lane
        row_hi = lane + 16

        def owait(b):
            pltpu.make_async_copy(
                out_v.at[b, :, pl.ds(0, BC)],
                out_hbm.at[0, :, pl.ds(0, BC)],
                osems[b],
            ).wait()

        def reduce_store(u, t, sub, b):
            @pl.when(u >= 2)
            def _():
                owait(b)

            def bag_body(bag, carry):
                a0 = rows_v[b, bag, pl.ds(0, 16)]
                a1 = rows_v[b, bag, pl.ds(16, 16)]
                for l in range(1, BAG):
                    a0 = a0 + rows_v[b, bag + l * BC, pl.ds(0, 16)]
                    a1 = a1 + rows_v[b, bag + l * BC, pl.ds(16, 16)]
                col = jnp.full((16,), 0, jnp.int32) + bag
                plsc.store_scatter(out_v.at[b], [row_lo, col], a0)
                plsc.store_scatter(out_v.at[b], [row_hi, col], a1)
                return carry

            lax.fori_loop(0, BC, bag_body, 0)
            pltpu.async_copy(
                out_v.at[b, :, pl.ds(0, BC)],
                out_hbm.at[t, :, pl.ds(b0s[sub], BC)],
                osems[b],
            )

        start(0, 0, 0)

        def pair_body(g, carry):
            for bpar in range(2):
                u = 2 * g + bpar
                nxt = 1 - bpar

                @pl.when(u + 1 < units_per_worker)
                def _():
                    start(g + bpar, nxt, nxt)

                wait(bpar)
                reduce_store(u, g, bpar, bpar)
            return carry

        lax.fori_loop(0, units_per_worker // 2, pair_body, 0)
        owait(0)
        owait(1)

    return k(table, idx_t)


def kernel(embedding_weights, sharded_sparse_features):
    N, T, D = embedding_weights.shape
    B, _, L = sharded_sparse_features.shape
    wt = embedding_weights.transpose(1, 2, 0)
    table = _sc_transpose(wt, N, T, 32)
    idx_t = sharded_sparse_features.astype(jnp.int32).transpose(1, 2, 0)
    out = _sc_lookup(table, idx_t, N, T, B, 32)
    return out.transpose(2, 0, 1)


# speedup vs baseline: 1.8935x; 1.0710x over previous
import functools

import jax
import jax.numpy as jnp
from jax import lax
from jax.experimental import pallas as pl
from jax.experimental.pallas import tpu as pltpu
from jax.experimental.pallas import tpu_sc as plsc

DIM = 32
BAG = 20
BC = 64
ROWS = BAG * BC
IDX_MINOR = 128
IDX_ROWS = ROWS // IDX_MINOR


NCH = 800


@functools.partial(jax.jit, static_argnums=(1, 2, 3))
def _sc_transpose(wt, N, T, num_workers):
    units = T * (N // NCH)
    upw = -(-units // num_workers)
    mesh = plsc.VectorSubcoreMesh(core_axis_name="c", subcore_axis_name="s")

    @functools.partial(
        pl.kernel,
        mesh=mesh,
        compiler_params=pltpu.CompilerParams(
            use_tc_tiling_on_sc=False, needs_layout_passes=False
        ),
        out_type=jax.ShapeDtypeStruct((T * N, DIM), jnp.float32),
        scratch_types=[
            pltpu.VMEM((2, DIM, NCH), jnp.float32),
            pltpu.VMEM((2, NCH, DIM + 1), jnp.float32),
            pltpu.SemaphoreType.DMA,
            pltpu.SemaphoreType.DMA,
            pltpu.SemaphoreType.DMA,
            pltpu.SemaphoreType.DMA,
        ],
    )
    def k(wt_hbm, out_hbm, slab_v, tr_v, sem0, sem1, osem0, osem1):
        sems = (sem0, sem1)
        osems = (osem0, osem1)
        wid = lax.axis_index("s") * 2 + lax.axis_index("c")
        u0 = wid * upw
        nchunks = N // NCH
        lane = jax.lax.iota(jnp.int32, 16)
        cols = [jnp.full((16,), d, jnp.int32) for d in range(DIM)]

        def start(u, b):
            t = u // nchunks
            n0 = (u % nchunks) * NCH
            pltpu.async_copy(
                wt_hbm.at[t, :, pl.ds(n0, NCH)], slab_v.at[b], sems[b]
            )

        def wait(b):
            pltpu.make_async_copy(
                wt_hbm.at[0, :, pl.ds(0, NCH)], slab_v.at[b], sems[b]
            ).wait()

        def owait(b):
            pltpu.make_async_copy(
                tr_v.at[b, :, pl.ds(0, DIM)],
                out_hbm.at[pl.ds(0, NCH), :],
                osems[b],
            ).wait()

        def process(u, b):
            @pl.when(u >= u0 + 2)
            def _():
                owait(b)

            def grp(g, carry):
                rows = lane + g * 16
                for d0 in range(0, DIM, 8):
                    vs = [slab_v[b, d, pl.ds(g * 16, 16)] for d in range(d0, d0 + 8)]
                    for i, d in enumerate(range(d0, d0 + 8)):
                        plsc.store_scatter(tr_v.at[b], [rows, cols[d]], vs[i])
                return carry

            lax.fori_loop(0, NCH // 16, grp, 0)
            t = u // nchunks
            n0 = (u % nchunks) * NCH
            pltpu.async_copy(
                tr_v.at[b, :, pl.ds(0, DIM)],
                out_hbm.at[pl.ds(t * N + n0, NCH), :],
                osems[b],
            )

        @pl.when(u0 < units)
        def _():
            start(u0, 0)

        def pair_body(g, carry):
            for bpar in range(2):
                u = u0 + 2 * g + bpar
                nxt = 1 - bpar

                @pl.when(u + 1 < jnp.minimum(u0 + upw, units))
                def _():
                    start(u + 1, nxt)

                @pl.when(u < units)
                def _():
                    wait(bpar)
                    process(u, bpar)
            return carry

        lax.fori_loop(0, upw // 2, pair_body, 0)

        owait(0)
        owait(1)

    return k(wt)


@functools.partial(jax.jit, static_argnums=(2, 3, 4, 5))
def _sc_lookup(table, idx_t, N, T, B, num_workers):
    units_per_worker = (T * B // BC) // num_workers
    mesh = plsc.VectorSubcoreMesh(core_axis_name="c", subcore_axis_name="s")

    @functools.partial(
        pl.kernel,
        mesh=mesh,
        compiler_params=pltpu.CompilerParams(
            use_tc_tiling_on_sc=False, needs_layout_passes=False
        ),
        out_type=jax.ShapeDtypeStruct((T, DIM, B), jnp.float32),
        scratch_types=[
            pltpu.VMEM((2, BAG, BC), jnp.int32),
            pltpu.VMEM((2, IDX_ROWS, IDX_MINOR), jnp.int32),
            pltpu.VMEM((2, ROWS, DIM), jnp.float32),
            pltpu.VMEM((2, DIM, BC + 1), jnp.float32),
            pltpu.SemaphoreType.DMA,
            pltpu.SemaphoreType.DMA,
            pltpu.SemaphoreType.DMA,
            pltpu.SemaphoreType.DMA,
        ],
    )
    def k(table_hbm, idx_hbm, out_hbm, idx_v, flat_v, rows_v, out_v, sem0, sem1, osem0, osem1):
        table2d = table_hbm
        sems = (sem0, sem1)
        osems = (osem0, osem1)
        wid = lax.axis_index("s") * 2 + lax.axis_index("c")
        b0s = (wid * 2 * BC, (wid * 2 + 1) * BC)

        def start(t, sub, b):
            pltpu.sync_copy(idx_hbm.at[t, :, pl.ds(b0s[sub], BC)], idx_v.at[b])
            for kk in range(ROWS // 16):
                v = idx_v[b, kk // 4, pl.ds((kk % 4) * 16, 16)]
                flat_v[b, kk // 8, pl.ds((kk % 8) * 16, 16)] = v + t * N
            for j in range(IDX_ROWS):
                pltpu.async_copy(
                    table2d.at[flat_v.at[b, j]],
                    rows_v.at[b, pl.ds(j * IDX_MINOR, IDX_MINOR)],
                    sems[b],
                )

        def wait(b):
            for j in range(IDX_ROWS):
                pltpu.make_async_copy(
                    table2d.at[flat_v.at[b, j]],
                    rows_v.at[b, pl.ds(j * IDX_MINOR, IDX_MINOR)],
                    sems[b],
                ).wait()

        lane = jax.lax.iota(jnp.int32, 16)
        row_lo = lane
        row_hi = lane + 16

        def owait(b):
            pltpu.make_async_copy(
                out_v.at[b, :, pl.ds(0, BC)],
                out_hbm.at[0, :, pl.ds(0, BC)],
                osems[b],
            ).wait()

        def reduce_store(u, t, sub, b):
            @pl.when(u >= 2)
            def _():
                owait(b)

            def bag_body(bag, carry):
                a0 = rows_v[b, bag, pl.ds(0, 16)]
                a1 = rows_v[b, bag, pl.ds(16, 16)]
                for l in range(1, BAG):
                    a0 = a0 + rows_v[b, bag + l * BC, pl.ds(0, 16)]
                    a1 = a1 + rows_v[b, bag + l * BC, pl.ds(16, 16)]
                col = jnp.full((16,), 0, jnp.int32) + bag
                plsc.store_scatter(out_v.at[b], [row_lo, col], a0)
                plsc.store_scatter(out_v.at[b], [row_hi, col], a1)
                return carry

            lax.fori_loop(0, BC, bag_body, 0)
            pltpu.async_copy(
                out_v.at[b, :, pl.ds(0, BC)],
                out_hbm.at[t, :, pl.ds(b0s[sub], BC)],
                osems[b],
            )

        start(0, 0, 0)

        def pair_body(g, carry):
            for bpar in range(2):
                u = 2 * g + bpar
                nxt = 1 - bpar

                @pl.when(u + 1 < units_per_worker)
                def _():
                    start(g + bpar, nxt, nxt)

                wait(bpar)
                reduce_store(u, g, bpar, bpar)
            return carry

        lax.fori_loop(0, units_per_worker // 2, pair_body, 0)
        owait(0)
        owait(1)

    return k(table, idx_t)


def kernel(embedding_weights, sharded_sparse_features):
    N, T, D = embedding_weights.shape
    B, _, L = sharded_sparse_features.shape
    wt = embedding_weights.transpose(1, 2, 0)
    table = _sc_transpose(wt, N, T, 32)
    idx_t = sharded_sparse_features.astype(jnp.int32).transpose(1, 2, 0)
    out = _sc_lookup(table, idx_t, N, T, B, 32)
    return out.transpose(2, 0, 1)
